# baseline (speedup 1.0000x reference)
import functools
import numpy as np
import jax
import jax.numpy as jnp
from jax import lax
from jax.experimental import pallas as pl
from jax.experimental.pallas import tpu as pltpu
from jax.experimental.pallas import tpu_sc as plsc

NA = 10000
NP = 320000
F = 128
G = 16
H = 8
HID = 256

NC = 2
NS = 16
NW = NC * NS
ROW = 125
CPT = NP // (NW * ROW)
PPT = NP // NW
NAP = 10240
RPS = NAP // NS


def _sc_segment_sum(idx2, gs, gv):
  mesh = plsc.VectorSubcoreMesh(core_axis_name="c", subcore_axis_name="s")

  @functools.partial(
      pl.kernel,
      out_type=(jax.ShapeDtypeStruct((NC, NAP, G), jnp.float32),
                jax.ShapeDtypeStruct((NC, NAP, 3 * G), jnp.float32)),
      mesh=mesh,
      scratch_types=[
          pltpu.VMEM((CPT, ROW), jnp.int32),
          pltpu.VMEM((ROW, G), jnp.float32),
          pltpu.VMEM((ROW, G), jnp.float32),
          pltpu.VMEM((ROW, 3 * G), jnp.float32),
          pltpu.VMEM((ROW, 3 * G), jnp.float32),
          pltpu.VMEM((RPS, G), jnp.float32),
          pltpu.VMEM((RPS, 3 * G), jnp.float32),
          pltpu.VMEM_SHARED((NAP, G), jnp.float32),
          pltpu.VMEM_SHARED((NAP, 3 * G), jnp.float32),
          pltpu.SemaphoreType.DMA,
          pltpu.SemaphoreType.DMA,
      ],
      compiler_params=pltpu.CompilerParams(use_tc_tiling_on_sc=False),
  )
  def seg(idx_hbm, gs_hbm, gv_hbm, ogs_hbm, ogv_hbm,
          idx_v, gs_v0, gs_v1, gv_v0, gv_v1, sgs_v, sgv_v,
          acc_gs, acc_gv, sem0, sem1):
    c = lax.axis_index("c")
    s = lax.axis_index("s")
    w = s * NC + c
    z16 = jnp.zeros((16,), jnp.float32)

    def zrow(r, carry):
      sgs_v[r] = z16
      for k in range(3):
        sgv_v[r, pl.ds(k * 16, 16)] = z16
      return carry
    lax.fori_loop(0, RPS, zrow, 0)
    pltpu.sync_copy(sgs_v, acc_gs.at[pl.ds(s * RPS, RPS)])
    pltpu.sync_copy(sgv_v, acc_gv.at[pl.ds(s * RPS, RPS)])
    plsc.subcore_barrier()

    pltpu.sync_copy(idx_hbm.at[pl.ds(w * CPT, CPT)], idx_v)

    def start_loads(j, gs_b, gv_b, sem):
      base = w * PPT + j * ROW
      pltpu.async_copy(gs_hbm.at[pl.ds(base, ROW)], gs_b, sem)
      pltpu.async_copy(gv_hbm.at[pl.ds(base, ROW)], gv_b, sem)

    def wait_loads(j, gs_b, gv_b, sem):
      base = w * PPT + j * ROW
      pltpu.make_async_copy(gs_hbm.at[pl.ds(base, ROW)], gs_b, sem).wait()
      pltpu.make_async_copy(gv_hbm.at[pl.ds(base, ROW)], gv_b, sem).wait()

    start_loads(0, gs_v0, gv_v0, sem0)

    def chunk2(t, carry):
      j0 = 2 * t
      j1 = 2 * t + 1
      wait_loads(j0, gs_v0, gv_v0, sem0)
      start_loads(j1, gs_v1, gv_v1, sem1)
      pltpu.sync_copy(gs_v0, acc_gs.at[idx_v.at[j0]], add=True)
      pltpu.sync_copy(gv_v0, acc_gv.at[idx_v.at[j0]], add=True)
      wait_loads(j1, gs_v1, gv_v1, sem1)

      @pl.when(t < CPT // 2 - 1)
      def _():
        start_loads(j0 + 2, gs_v0, gv_v0, sem0)

      pltpu.sync_copy(gs_v1, acc_gs.at[idx_v.at[j1]], add=True)
      pltpu.sync_copy(gv_v1, acc_gv.at[idx_v.at[j1]], add=True)
      return carry
    lax.fori_loop(0, CPT // 2, chunk2, 0)
    plsc.subcore_barrier()

    pltpu.sync_copy(acc_gs.at[pl.ds(s * RPS, RPS)], sgs_v)
    pltpu.sync_copy(sgs_v, ogs_hbm.at[c, pl.ds(s * RPS, RPS)])
    pltpu.sync_copy(acc_gv.at[pl.ds(s * RPS, RPS)], sgv_v)
    pltpu.sync_copy(sgv_v, ogv_hbm.at[c, pl.ds(s * RPS, RPS)])

  return seg(idx2, gs, gv)


_BB = 1000


def _gelu_exact(x):
  return 0.5 * x * (1.0 + lax.erf(x * np.float32(0.7071067811865476)))


def _tc_body(emb_r, q_r, pgs_r, pgv_r, wgsT_r, agh2_r, e16_r, s128_r,
             w1aT_r, w1bT_r, w1cT_r, b1_r, w2T_r, b2_r,
             w3aT_r, w3bT_r, b3a_r, b3b_r, oa_r, oqf_r):
  f32 = jnp.float32
  e = emb_r[...]
  gsum = pgs_r[0] + pgs_r[1]
  mapped = jnp.dot(gsum, wgsT_r[...], preferred_element_type=f32)
  t = jnp.dot(e, agh2_r[...], preferred_element_type=f32)
  gvsum = pgv_r[0] + pgv_r[1]
  sq = None
  for d in range(3):
    gvd = gvsum[:, d * G:(d + 1) * G]
    gexp = jnp.dot(gvd, e16_r[...], preferred_element_type=f32)
    ad = jnp.dot(gexp * t, s128_r[...], preferred_element_type=f32)
    sq = ad * ad if sq is None else sq + ad * ad
  vec = jnp.sqrt(sq)
  radial = e * mapped
  radial_q = q_r[...] * mapped
  h1 = (jnp.dot(radial, w1aT_r[...], preferred_element_type=f32)
        + jnp.dot(vec, w1bT_r[...], preferred_element_type=f32)
        + jnp.dot(radial_q, w1cT_r[...], preferred_element_type=f32)
        + b1_r[...])
  h1 = _gelu_exact(h1)
  h2 = _gelu_exact(jnp.dot(h1, w2T_r[...], preferred_element_type=f32)
                   + b2_r[...])
  oa_r[...] = jnp.dot(h2, w3aT_r[...], preferred_element_type=f32) + b3a_r[...]
  oqf_r[...] = jnp.dot(h2, w3bT_r[...], preferred_element_type=f32) + b3b_r[...]


def _tc_dense(emb, q, pgs, pgv, wgsT, agh2, e16, s128,
              w1aT, w1bT, w1cT, b1r, w2T, b2r, w3aT, w3bT, b3a, b3b):
  nblk = NA // _BB
  full = lambda a: pl.BlockSpec(a.shape, lambda i: tuple(0 for _ in a.shape))
  return pl.pallas_call(
      _tc_body,
      grid=(nblk,),
      in_specs=[
          pl.BlockSpec((_BB, F), lambda i: (i, 0)),
          pl.BlockSpec((_BB, 1), lambda i: (i, 0)),
          pl.BlockSpec((NC, _BB, G), lambda i: (0, i, 0)),
          pl.BlockSpec((NC, _BB, 3 * G), lambda i: (0, i, 0)),
          full(wgsT), full(agh2), full(e16), full(s128),
          full(w1aT), full(w1bT), full(w1cT), full(b1r),
          full(w2T), full(b2r), full(w3aT), full(w3bT), full(b3a), full(b3b),
      ],
      out_specs=[
          pl.BlockSpec((_BB, F), lambda i: (i, 0)),
          pl.BlockSpec((_BB, H), lambda i: (i, 0)),
      ],
      out_shape=[
          jax.ShapeDtypeStruct((NA, F), jnp.float32),
          jax.ShapeDtypeStruct((NA, H), jnp.float32),
      ],
      compiler_params=pltpu.CompilerParams(
          dimension_semantics=("parallel",)),
  )(emb, q, pgs, pgv, wgsT, agh2, e16, s128,
    w1aT, w1bT, w1cT, b1r, w2T, b2r, w3aT, w3bT, b3a, b3b)


_E16 = np.repeat(np.eye(G, dtype=np.float32), H, axis=1)
_S128 = np.tile(np.eye(H, dtype=np.float32), (G, 1))


def kernel(atomic_embedding, partial_charges, pair_indices, gs, gv, agh,
           W_gs, W1, b1, W2, b2, W3, b3):
  idx2 = pair_indices[1].reshape(NW * CPT, ROW)
  gv48 = jnp.concatenate([gv[:, 0], gv[:, 1], gv[:, 2]], axis=1)
  pgs, pgv = _sc_segment_sum(idx2, gs, gv48)

  wgsT = W_gs.T
  agh2 = agh.reshape(F, G * H)
  w1aT = W1[:, 0:F].T
  w1bT = W1[:, F:F + H].T
  w1cT = W1[:, F + H:2 * F + H].T
  w2T = W2.T
  w3aT = W3[2:].T
  w3bT = jnp.zeros((HID, H), jnp.float32).at[:, 0:2].set(W3[0:2].T)
  b1r = b1.reshape(1, HID)
  b2r = b2.reshape(1, HID)
  b3a = b3[2:].reshape(1, F)
  b3b = jnp.zeros((1, H), jnp.float32).at[0, 0:2].set(b3[0:2])

  oa, oqf = _tc_dense(atomic_embedding, partial_charges, pgs, pgv,
                      wgsT, agh2, jnp.asarray(_E16), jnp.asarray(_S128),
                      w1aT, w1bT, w1cT, b1r, w2T, b2r, w3aT, w3bT, b3a, b3b)
  return (oa, oqf[:, 0:1], oqf[:, 1:2])


# speedup vs baseline: 1.1671x; 1.1671x over previous
import functools
import numpy as np
import jax
import jax.numpy as jnp
from jax import lax
from jax.experimental import pallas as pl
from jax.experimental.pallas import tpu as pltpu
from jax.experimental.pallas import tpu_sc as plsc

NA = 10000
NP = 320000
F = 128
G = 16
H = 8
HID = 256

NC = 2
NS = 16
NW = NC * NS
ROW = 125
CPT = NP // (NW * ROW)
PPT = NP // NW
NAP = 10240
RPS = NAP // NS


def _sc_segment_sum(idx2, gs, gv):
  mesh = plsc.VectorSubcoreMesh(core_axis_name="c", subcore_axis_name="s")

  @functools.partial(
      pl.kernel,
      out_type=(jax.ShapeDtypeStruct((NC, NAP, G), jnp.float32),
                jax.ShapeDtypeStruct((NC, NAP, 3 * G), jnp.float32)),
      mesh=mesh,
      scratch_types=[
          pltpu.VMEM((CPT, ROW), jnp.int32),
          pltpu.VMEM((ROW, G), jnp.float32),
          pltpu.VMEM((ROW, G), jnp.float32),
          pltpu.VMEM((ROW, 3 * G), jnp.float32),
          pltpu.VMEM((ROW, 3 * G), jnp.float32),
          pltpu.VMEM((RPS, G), jnp.float32),
          pltpu.VMEM((RPS, 3 * G), jnp.float32),
          pltpu.VMEM_SHARED((NAP, G), jnp.float32),
          pltpu.VMEM_SHARED((NAP, 3 * G), jnp.float32),
          pltpu.SemaphoreType.DMA,
          pltpu.SemaphoreType.DMA,
      ],
      compiler_params=pltpu.CompilerParams(use_tc_tiling_on_sc=False),
  )
  def seg(idx_hbm, gs_hbm, gv_hbm, ogs_hbm, ogv_hbm,
          idx_v, gs_v0, gs_v1, gv_v0, gv_v1, sgs_v, sgv_v,
          acc_gs, acc_gv, sem0, sem1):
    c = lax.axis_index("c")
    s = lax.axis_index("s")
    w = s * NC + c
    z16 = jnp.zeros((16,), jnp.float32)

    def zrow(r, carry):
      sgs_v[r] = z16
      for k in range(3):
        sgv_v[r, pl.ds(k * 16, 16)] = z16
      return carry
    lax.fori_loop(0, RPS, zrow, 0)
    pltpu.sync_copy(sgs_v, acc_gs.at[pl.ds(s * RPS, RPS)])
    pltpu.sync_copy(sgv_v, acc_gv.at[pl.ds(s * RPS, RPS)])
    plsc.subcore_barrier()

    pltpu.sync_copy(idx_hbm.at[pl.ds(w * CPT, CPT)], idx_v)

    def start_loads(j, gs_b, gv_b, sem):
      base = w * PPT + j * ROW
      pltpu.async_copy(gs_hbm.at[pl.ds(base, ROW)], gs_b, sem)
      pltpu.async_copy(gv_hbm.at[pl.ds(base, ROW)], gv_b, sem)

    def wait_loads(j, gs_b, gv_b, sem):
      base = w * PPT + j * ROW
      pltpu.make_async_copy(gs_hbm.at[pl.ds(base, ROW)], gs_b, sem).wait()
      pltpu.make_async_copy(gv_hbm.at[pl.ds(base, ROW)], gv_b, sem).wait()

    start_loads(0, gs_v0, gv_v0, sem0)

    def chunk2(t, carry):
      j0 = 2 * t
      j1 = 2 * t + 1
      wait_loads(j0, gs_v0, gv_v0, sem0)
      start_loads(j1, gs_v1, gv_v1, sem1)
      pltpu.sync_copy(gs_v0, acc_gs.at[idx_v.at[j0]], add=True)
      pltpu.sync_copy(gv_v0, acc_gv.at[idx_v.at[j0]], add=True)
      wait_loads(j1, gs_v1, gv_v1, sem1)

      @pl.when(t < CPT // 2 - 1)
      def _():
        start_loads(j0 + 2, gs_v0, gv_v0, sem0)

      pltpu.sync_copy(gs_v1, acc_gs.at[idx_v.at[j1]], add=True)
      pltpu.sync_copy(gv_v1, acc_gv.at[idx_v.at[j1]], add=True)
      return carry
    lax.fori_loop(0, CPT // 2, chunk2, 0)
    plsc.subcore_barrier()

    pltpu.sync_copy(acc_gs.at[pl.ds(s * RPS, RPS)], sgs_v)
    pltpu.sync_copy(sgs_v, ogs_hbm.at[c, pl.ds(s * RPS, RPS)])
    pltpu.sync_copy(acc_gv.at[pl.ds(s * RPS, RPS)], sgv_v)
    pltpu.sync_copy(sgv_v, ogv_hbm.at[c, pl.ds(s * RPS, RPS)])

  return seg(idx2, gs, gv)


_BB = 1000


def _gelu_exact(x):
  return 0.5 * x * (1.0 + lax.erf(x * np.float32(0.7071067811865476)))


def _tc_body(emb_r, q_r, pgs_r, pgv_r, wgsT_r, agh2_r, e16_r, s128_r,
             w1aT_r, w1bT_r, w1cT_r, b1_r, w2T_r, b2_r,
             w3aT_r, w3bT_r, b3a_r, b3b_r, oa_r, oqf_r):
  f32 = jnp.float32
  e = emb_r[...]
  gsum = pgs_r[0] + pgs_r[1]
  mapped = jnp.dot(gsum, wgsT_r[...], preferred_element_type=f32)
  t = jnp.dot(e, agh2_r[...], preferred_element_type=f32)
  gvsum = pgv_r[0] + pgv_r[1]
  sq = None
  for d in range(3):
    gvd = gvsum[:, d * G:(d + 1) * G]
    gexp = jnp.dot(gvd, e16_r[...], preferred_element_type=f32)
    ad = jnp.dot(gexp * t, s128_r[...], preferred_element_type=f32)
    sq = ad * ad if sq is None else sq + ad * ad
  vec = jnp.sqrt(sq)
  radial = e * mapped
  radial_q = q_r[...] * mapped
  h1 = (jnp.dot(radial, w1aT_r[...], preferred_element_type=f32)
        + jnp.dot(vec, w1bT_r[...], preferred_element_type=f32)
        + jnp.dot(radial_q, w1cT_r[...], preferred_element_type=f32)
        + b1_r[...])
  h1 = _gelu_exact(h1)
  h2 = _gelu_exact(jnp.dot(h1, w2T_r[...], preferred_element_type=f32)
                   + b2_r[...])
  oa_r[...] = jnp.dot(h2, w3aT_r[...], preferred_element_type=f32) + b3a_r[...]
  oqf_r[...] = jnp.dot(h2, w3bT_r[...], preferred_element_type=f32) + b3b_r[...]


def _tc_dense(emb, q, pgs, pgv, wgsT, agh2, e16, s128,
              w1aT, w1bT, w1cT, b1r, w2T, b2r, w3aT, w3bT, b3a, b3b):
  nblk = NA // _BB
  full = lambda a: pl.BlockSpec(a.shape, lambda i: tuple(0 for _ in a.shape))
  return pl.pallas_call(
      _tc_body,
      grid=(nblk,),
      in_specs=[
          pl.BlockSpec((_BB, F), lambda i: (i, 0)),
          pl.BlockSpec((_BB, 1), lambda i: (i, 0)),
          pl.BlockSpec((NC, _BB, G), lambda i: (0, i, 0)),
          pl.BlockSpec((NC, _BB, 3 * G), lambda i: (0, i, 0)),
          full(wgsT), full(agh2), full(e16), full(s128),
          full(w1aT), full(w1bT), full(w1cT), full(b1r),
          full(w2T), full(b2r), full(w3aT), full(w3bT), full(b3a), full(b3b),
      ],
      out_specs=[
          pl.BlockSpec((_BB, F), lambda i: (i, 0)),
          pl.BlockSpec((_BB, H), lambda i: (i, 0)),
      ],
      out_shape=[
          jax.ShapeDtypeStruct((NA, F), jnp.float32),
          jax.ShapeDtypeStruct((NA, H), jnp.float32),
      ],
      compiler_params=pltpu.CompilerParams(
          dimension_semantics=("parallel",)),
  )(emb, q, pgs, pgv, wgsT, agh2, e16, s128,
    w1aT, w1bT, w1cT, b1r, w2T, b2r, w3aT, w3bT, b3a, b3b)


_E16 = np.repeat(np.eye(G, dtype=np.float32), H, axis=1)
_S128 = np.tile(np.eye(H, dtype=np.float32), (G, 1))


def kernel(atomic_embedding, partial_charges, pair_indices, gs, gv, agh,
           W_gs, W1, b1, W2, b2, W3, b3):
  idx2 = pair_indices[1].reshape(NW * CPT, ROW)
  pgs, pgv = _sc_segment_sum(idx2, gs, gv.reshape(NP, 3 * G))

  wgsT = W_gs.T
  agh2 = agh.reshape(F, G * H)
  w1aT = W1[:, 0:F].T
  w1bT = W1[:, F:F + H].T
  w1cT = W1[:, F + H:2 * F + H].T
  w2T = W2.T
  w3aT = W3[2:].T
  w3bT = jnp.zeros((HID, H), jnp.float32).at[:, 0:2].set(W3[0:2].T)
  b1r = b1.reshape(1, HID)
  b2r = b2.reshape(1, HID)
  b3a = b3[2:].reshape(1, F)
  b3b = jnp.zeros((1, H), jnp.float32).at[0, 0:2].set(b3[0:2])

  oa, oqf = _tc_dense(atomic_embedding, partial_charges, pgs, pgv,
                      wgsT, agh2, jnp.asarray(_E16), jnp.asarray(_S128),
                      w1aT, w1bT, w1cT, b1r, w2T, b2r, w3aT, w3bT, b3a, b3b)
  return (oa, oqf[:, 0:1], oqf[:, 1:2])
